# Initial kernel scaffold; baseline (speedup 1.0000x reference)
import jax
import jax.numpy as jnp
from jax.experimental import pallas as pl


def kernel(x, perm):
    raise NotImplementedError("write your pallas kernel here")


# speedup vs baseline: 1.0871x; 1.0871x over previous
import functools

import jax
import jax.numpy as jnp
from jax import lax
from jax.experimental import pallas as pl
from jax.experimental.pallas import tpu as pltpu
from jax.experimental.pallas import tpu_sc as plsc

_B = 32
_C = 192
_HW = 56 * 56
_ROWS = _B * _C

_info = plsc.get_sparse_core_info()
_NC = _info.num_cores
_NS = _info.num_subcores
_NW = _NC * _NS
_RPW = _ROWS // _NW
_CHUNK = 16
_NCHUNK = _RPW // _CHUNK

_mesh = plsc.VectorSubcoreMesh(core_axis_name="c", subcore_axis_name="s")


@functools.partial(
    pl.kernel,
    mesh=_mesh,
    compiler_params=pltpu.CompilerParams(use_tc_tiling_on_sc=False),
    out_type=jax.ShapeDtypeStruct((_ROWS, _HW), jnp.float32),
    scratch_types=[
        pltpu.VMEM((_NCHUNK, _CHUNK), jnp.int32),
        pltpu.VMEM((2, _CHUNK, _HW), jnp.float32),
        pltpu.SemaphoreType.DMA,
        pltpu.SemaphoreType.DMA,
        pltpu.SemaphoreType.DMA,
        pltpu.SemaphoreType.DMA,
    ],
)
def _permute_rows(x_hbm, idx_hbm, out_hbm, idx_v, buf_v, g0, g1, s0, s1):
    wid = lax.axis_index("s") * _NC + lax.axis_index("c")
    base = wid * _RPW
    pltpu.sync_copy(idx_hbm.at[wid], idx_v)

    gsems = [g0, g1]
    ssems = [s0, s1]

    def gather_start(i, slot):
        return pltpu.async_copy(
            x_hbm.at[idx_v.at[i]], buf_v.at[slot], gsems[slot]
        )

    def store_start(i, slot):
        return pltpu.async_copy(
            buf_v.at[slot],
            out_hbm.at[pl.ds(base + i * _CHUNK, _CHUNK)],
            ssems[slot],
        )

    gathers = [gather_start(0, 0), gather_start(1, 1)]
    stores = [None, None]
    for i in range(_NCHUNK):
        slot = i & 1
        gathers[slot].wait()
        stores[slot] = store_start(i, slot)
        if i + 2 < _NCHUNK:
            stores[slot].wait()
            gathers[slot] = gather_start(i + 2, slot)
    stores[0].wait()
    stores[1].wait()


def kernel(x, perm):
    perm = perm.astype(jnp.int32)
    idx = (jnp.arange(_B, dtype=jnp.int32)[:, None] * _C + perm[None, :])
    idx = idx.reshape(_NW, _NCHUNK, _CHUNK)
    x_rows = x.reshape(_ROWS, _HW)
    out = _permute_rows(x_rows, idx)
    return out.reshape(_B, _C, 56, 56)
